# baseline (speedup 1.0000x reference)
import math

import jax
import jax.numpy as jnp
from jax.experimental import pallas as pl

B, S, D = 1, 2048, 1024
H, E, K = 16, 8, 2
DH = D // H
HE = H * E
SCALE = (1.0 / math.sqrt(DH)) ** 0.5

TS = 256
TQ = 256


def _top2_gate(logits):
    ts = logits.shape[0]
    s = jax.nn.sigmoid(logits).reshape(ts, H, E)
    m1 = jnp.max(s, axis=-1, keepdims=True)
    eq = s >= m1
    idx = jax.lax.broadcasted_iota(jnp.int32, s.shape, 2)
    first_idx = jnp.min(jnp.where(eq, idx, E), axis=-1, keepdims=True)
    first = idx == first_idx
    s_masked = jnp.where(first, -1.0, s)
    m2 = jnp.max(s_masked, axis=-1, keepdims=True)
    g = jnp.where(s >= m2, s, 0.0)
    return g.reshape(ts, HE)


def _proj_kernel(xq_ref, xk_ref, xv_ref, wq_ref, wk_ref, sv_ref, so_ref,
                 wv_ref, q_ref, k_ref, v_ref, go_ref):
    xq = xq_ref[...]
    xk = xk_ref[...]
    q = jax.lax.dot_general(
        xq.astype(jnp.bfloat16), wq_ref[...], (((1,), (1,)), ((), ())),
        preferred_element_type=jnp.float32) * SCALE
    k = jax.lax.dot_general(
        xk.astype(jnp.bfloat16), wk_ref[...], (((1,), (1,)), ((), ())),
        preferred_element_type=jnp.float32) * SCALE
    q_ref[...] = q
    k_ref[...] = k

    lv = jax.lax.dot_general(xk, sv_ref[...], (((1,), (1,)), ((), ())),
                             preferred_element_type=jnp.float32)
    lo = jax.lax.dot_general(xq, so_ref[...], (((1,), (1,)), ((), ())),
                             preferred_element_type=jnp.float32)
    gv = _top2_gate(lv)
    go_ref[...] = _top2_gate(lo)

    vall = jax.lax.dot_general(
        xv_ref[...].astype(jnp.bfloat16), wv_ref[...], (((1,), (0,)), ((), ())),
        preferred_element_type=jnp.float32)
    v4 = vall.reshape(TS, H, E, DH)
    g4 = gv.reshape(TS, H, E, 1)
    v_ref[...] = jnp.sum(v4 * g4, axis=2).reshape(TS, H * DH)


def _attn_kernel(q_ref, k_ref, v_ref, o_ref):
    q = q_ref[0].astype(jnp.bfloat16)
    k = k_ref[0].astype(jnp.bfloat16)
    s = jax.lax.dot_general(q, k, (((1,), (1,)), ((), ())),
                            preferred_element_type=jnp.float32)
    m = jnp.max(s, axis=-1, keepdims=True)
    p = jnp.exp(s - m)
    p = p / jnp.sum(p, axis=-1, keepdims=True)
    o = jax.lax.dot_general(p.astype(jnp.bfloat16), v_ref[0].astype(jnp.bfloat16),
                            (((1,), (0,)), ((), ())),
                            preferred_element_type=jnp.float32)
    o_ref[0] = o


def _out_kernel(r_ref, g_ref, wo_ref, o_ref):
    r = r_ref[...]
    g = g_ref[...]
    y = (r.reshape(TS, H, 1, DH) * g.reshape(TS, H, E, 1)).reshape(TS, HE * DH)
    o_ref[...] = jax.lax.dot_general(
        y.astype(jnp.bfloat16), wo_ref[...], (((1,), (0,)), ((), ())),
        preferred_element_type=jnp.float32)


def kernel(q_src, k_src, v_src, Wq, Wk, Wv, Wo, sel_v, sel_o):
    xq = q_src.reshape(S, D)
    xk = k_src.reshape(S, D)
    xv = v_src.reshape(S, D)
    wq = Wq.astype(jnp.bfloat16)
    wk = Wk.astype(jnp.bfloat16)
    wv_flat = Wv.astype(jnp.bfloat16).transpose(1, 0, 2).reshape(D, HE * DH)
    wo_flat = Wo.astype(jnp.bfloat16).reshape(HE * DH, D)

    f32 = jnp.float32
    q, k, v, gate_o = pl.pallas_call(
        _proj_kernel,
        grid=(S // TS,),
        in_specs=[
            pl.BlockSpec((TS, D), lambda i: (i, 0)),
            pl.BlockSpec((TS, D), lambda i: (i, 0)),
            pl.BlockSpec((TS, D), lambda i: (i, 0)),
            pl.BlockSpec((D, D), lambda i: (0, 0)),
            pl.BlockSpec((D, D), lambda i: (0, 0)),
            pl.BlockSpec((HE, D), lambda i: (0, 0)),
            pl.BlockSpec((HE, D), lambda i: (0, 0)),
            pl.BlockSpec((D, HE * DH), lambda i: (0, 0)),
        ],
        out_specs=[
            pl.BlockSpec((TS, D), lambda i: (i, 0)),
            pl.BlockSpec((TS, D), lambda i: (i, 0)),
            pl.BlockSpec((TS, D), lambda i: (i, 0)),
            pl.BlockSpec((TS, HE), lambda i: (i, 0)),
        ],
        out_shape=[
            jax.ShapeDtypeStruct((S, D), f32),
            jax.ShapeDtypeStruct((S, D), f32),
            jax.ShapeDtypeStruct((S, D), f32),
            jax.ShapeDtypeStruct((S, HE), f32),
        ],
    )(xq, xk, xv, wq, wk, sel_v, sel_o, wv_flat)

    q3 = q.reshape(S, H, DH).transpose(1, 0, 2)
    k3 = k.reshape(S, H, DH).transpose(1, 0, 2)
    v3 = v.reshape(S, H, DH).transpose(1, 0, 2)
    res3 = pl.pallas_call(
        _attn_kernel,
        grid=(H, S // TQ),
        in_specs=[
            pl.BlockSpec((1, TQ, DH), lambda h, i: (h, i, 0)),
            pl.BlockSpec((1, S, DH), lambda h, i: (h, 0, 0)),
            pl.BlockSpec((1, S, DH), lambda h, i: (h, 0, 0)),
        ],
        out_specs=pl.BlockSpec((1, TQ, DH), lambda h, i: (h, i, 0)),
        out_shape=jax.ShapeDtypeStruct((H, S, DH), f32),
    )(q3, k3, v3)
    res = res3.transpose(1, 0, 2).reshape(S, D)

    out = pl.pallas_call(
        _out_kernel,
        grid=(S // TS,),
        in_specs=[
            pl.BlockSpec((TS, D), lambda i: (i, 0)),
            pl.BlockSpec((TS, HE), lambda i: (i, 0)),
            pl.BlockSpec((HE * DH, D), lambda i: (0, 0)),
        ],
        out_specs=pl.BlockSpec((TS, D), lambda i: (i, 0)),
        out_shape=jax.ShapeDtypeStruct((S, D), f32),
    )(res, gate_o, wo_flat)

    return out.reshape(B, S, D)


# speedup vs baseline: 1.3163x; 1.3163x over previous
import math

import jax
import jax.numpy as jnp
from jax.experimental import pallas as pl
from jax.experimental.pallas import tpu as pltpu

B, S, D = 1, 2048, 1024
H, E, K = 16, 8, 2
DH = D // H
HE = H * E
SCALE = (1.0 / math.sqrt(DH)) ** 0.5

TS = 256
TQ = 256


def _top2_gate_t(logits_t):
    ts = logits_t.shape[-1]
    s = jax.nn.sigmoid(logits_t).reshape(H, E, ts)
    m1 = jnp.max(s, axis=1, keepdims=True)
    eq = s >= m1
    idx = jax.lax.broadcasted_iota(jnp.int32, s.shape, 1)
    first_idx = jnp.min(jnp.where(eq, idx, E), axis=1, keepdims=True)
    s_masked = jnp.where(idx == first_idx, -1.0, s)
    m2 = jnp.max(s_masked, axis=1, keepdims=True)
    return jnp.where(s >= m2, s, 0.0)


def _proj_kernel(xq_ref, xk_ref, xv_ref, wq_ref, wk_ref, sv_ref, so_ref,
                 wv_ref, q_ref, k_ref, v_ref, go_ref):
    xq = xq_ref[...]
    xk = xk_ref[...]
    bf = jnp.bfloat16
    q_ref[...] = jax.lax.dot_general(
        wq_ref[...], xq.astype(bf), (((1,), (1,)), ((), ())),
        preferred_element_type=jnp.float32).astype(bf)
    k_ref[...] = jax.lax.dot_general(
        wk_ref[...], xk.astype(bf), (((1,), (1,)), ((), ())),
        preferred_element_type=jnp.float32).astype(bf)

    lv = jax.lax.dot_general(sv_ref[...], xk, (((1,), (1,)), ((), ())),
                             preferred_element_type=jnp.float32)
    lo = jax.lax.dot_general(so_ref[...], xq, (((1,), (1,)), ((), ())),
                             preferred_element_type=jnp.float32)
    gv = _top2_gate_t(lv)
    go_ref[...] = _top2_gate_t(lo).reshape(HE, TS)

    vall = jax.lax.dot_general(
        wv_ref[...], xv_ref[...].astype(bf), (((0,), (1,)), ((), ())),
        preferred_element_type=jnp.float32)
    v4 = vall.reshape(H, E, DH, TS)
    v_ref[...] = jnp.sum(v4 * gv.reshape(H, E, 1, TS), axis=1
                         ).reshape(H * DH, TS).astype(bf)


def _attn_out_kernel(q_ref, k_ref, v_ref, go_ref, wo_ref, o_ref):
    h = pl.program_id(1)
    q = q_ref[...]
    k = k_ref[...]
    s = jax.lax.dot_general(q, k, (((0,), (0,)), ((), ())),
                            preferred_element_type=jnp.float32)
    m = jnp.max(s, axis=-1, keepdims=True)
    p = jnp.exp(s - m)
    p = p / jnp.sum(p, axis=-1, keepdims=True)
    res = jax.lax.dot_general(p.astype(jnp.bfloat16), v_ref[...],
                              (((1,), (1,)), ((), ())),
                              preferred_element_type=jnp.float32)
    g = go_ref[...].T
    y = (g[:, :, None] * res[:, None, :]).reshape(TQ, E * DH)
    wo_h = wo_ref[pl.ds(h * (E * DH), E * DH), :]
    partial = jax.lax.dot_general(y.astype(jnp.bfloat16), wo_h,
                                  (((1,), (0,)), ((), ())),
                                  preferred_element_type=jnp.float32)

    @pl.when(h == 0)
    def _():
        o_ref[...] = partial

    @pl.when(h > 0)
    def _():
        o_ref[...] += partial


def kernel(q_src, k_src, v_src, Wq, Wk, Wv, Wo, sel_v, sel_o):
    xq = q_src.reshape(S, D)
    xk = k_src.reshape(S, D)
    xv = v_src.reshape(S, D)
    bf = jnp.bfloat16
    wq = (Wq * SCALE).astype(bf)
    wk = (Wk * SCALE).astype(bf)
    wv_flat = Wv.astype(bf).transpose(1, 0, 2).reshape(D, HE * DH)
    wo_flat = Wo.astype(bf).reshape(HE * DH, D)

    f32 = jnp.float32
    q_t, k_t, v_t, go_t = pl.pallas_call(
        _proj_kernel,
        grid=(S // TS,),
        in_specs=[
            pl.BlockSpec((TS, D), lambda i: (i, 0)),
            pl.BlockSpec((TS, D), lambda i: (i, 0)),
            pl.BlockSpec((TS, D), lambda i: (i, 0)),
            pl.BlockSpec((D, D), lambda i: (0, 0)),
            pl.BlockSpec((D, D), lambda i: (0, 0)),
            pl.BlockSpec((HE, D), lambda i: (0, 0)),
            pl.BlockSpec((HE, D), lambda i: (0, 0)),
            pl.BlockSpec((D, HE * DH), lambda i: (0, 0)),
        ],
        out_specs=[
            pl.BlockSpec((D, TS), lambda i: (0, i)),
            pl.BlockSpec((D, TS), lambda i: (0, i)),
            pl.BlockSpec((D, TS), lambda i: (0, i)),
            pl.BlockSpec((HE, TS), lambda i: (0, i)),
        ],
        out_shape=[
            jax.ShapeDtypeStruct((D, S), bf),
            jax.ShapeDtypeStruct((D, S), bf),
            jax.ShapeDtypeStruct((D, S), bf),
            jax.ShapeDtypeStruct((HE, S), f32),
        ],
        compiler_params=pltpu.CompilerParams(
            dimension_semantics=("parallel",)),
    )(xq, xk, xv, wq, wk, sel_v, sel_o, wv_flat)

    out = pl.pallas_call(
        _attn_out_kernel,
        grid=(S // TQ, H),
        in_specs=[
            pl.BlockSpec((DH, TQ), lambda i, h: (h, i)),
            pl.BlockSpec((DH, S), lambda i, h: (h, 0)),
            pl.BlockSpec((DH, S), lambda i, h: (h, 0)),
            pl.BlockSpec((E, TQ), lambda i, h: (h, i)),
            pl.BlockSpec((HE * DH, D), lambda i, h: (0, 0)),
        ],
        out_specs=pl.BlockSpec((TQ, D), lambda i, h: (i, 0)),
        out_shape=jax.ShapeDtypeStruct((S, D), f32),
        compiler_params=pltpu.CompilerParams(
            dimension_semantics=("parallel", "arbitrary")),
    )(q_t, k_t, v_t, go_t, wo_flat)

    return out.reshape(B, S, D)


# speedup vs baseline: 1.8079x; 1.3735x over previous
import math

import jax
import jax.numpy as jnp
from jax.experimental import pallas as pl
from jax.experimental.pallas import tpu as pltpu

B, S, D = 1, 2048, 1024
H, E, K = 16, 8, 2
DH = D // H
HE = H * E
SCALE = (1.0 / math.sqrt(DH)) ** 0.5

TS = 256
TQ = 256


def _top2_gate_t(logits_t):
    ts = logits_t.shape[-1]
    s = jax.nn.sigmoid(logits_t).reshape(H, E, ts)
    m1 = jnp.max(s, axis=1, keepdims=True)
    eq = s >= m1
    idx = jax.lax.broadcasted_iota(jnp.int32, s.shape, 1)
    first_idx = jnp.min(jnp.where(eq, idx, E), axis=1, keepdims=True)
    s_masked = jnp.where(idx == first_idx, -1.0, s)
    m2 = jnp.max(s_masked, axis=1, keepdims=True)
    return jnp.where(s >= m2, s, 0.0)


def _proj_kernel(xq_ref, xk_ref, xv_ref, wq_ref, wk_ref, sv_ref, so_ref,
                 wv_ref, q_ref, k_ref, v_ref, go_ref):
    xq = xq_ref[...]
    xk = xk_ref[...]
    bf = jnp.bfloat16
    q_ref[...] = jax.lax.dot_general(
        wq_ref[...], xq.astype(bf), (((1,), (1,)), ((), ())),
        preferred_element_type=jnp.float32).astype(bf)
    k_ref[...] = jax.lax.dot_general(
        wk_ref[...], xk.astype(bf), (((1,), (1,)), ((), ())),
        preferred_element_type=jnp.float32).astype(bf)

    lv = jax.lax.dot_general(sv_ref[...], xk, (((1,), (1,)), ((), ())),
                             preferred_element_type=jnp.float32)
    lo = jax.lax.dot_general(so_ref[...], xq, (((1,), (1,)), ((), ())),
                             preferred_element_type=jnp.float32)
    gv = _top2_gate_t(lv)
    go_ref[...] = _top2_gate_t(lo).reshape(HE, TS)

    vall = jax.lax.dot_general(
        wv_ref[...], xv_ref[...].astype(bf), (((0,), (1,)), ((), ())),
        preferred_element_type=jnp.float32)
    v4 = vall.reshape(H, E, DH, TS)
    v_ref[...] = jnp.sum(v4 * gv.reshape(H, E, 1, TS), axis=1
                         ).reshape(H * DH, TS).astype(bf)


def _one_head(q, k, v, g_raw):
    s = jax.lax.dot_general(q, k, (((0,), (0,)), ((), ())),
                            preferred_element_type=jnp.float32)
    m = jnp.max(s, axis=-1, keepdims=True)
    p = jnp.exp(s - m)
    denom = jnp.sum(p, axis=-1, keepdims=True)
    res_t = jax.lax.dot_general(v, p.astype(jnp.bfloat16),
                                (((1,), (1,)), ((), ())),
                                preferred_element_type=jnp.float32)
    g = g_raw * (1.0 / denom).T
    return (g.reshape(E, 1, TQ) * res_t.reshape(1, DH, TQ)).reshape(E * DH, TQ)


def _attn_out_kernel(q_ref, k_ref, v_ref, go_ref, wo_ref, o_ref):
    j = pl.program_id(1)
    q2 = q_ref[...]
    k2 = k_ref[...]
    v2 = v_ref[...]
    g2 = go_ref[...]
    y_a = _one_head(q2[:DH], k2[:DH], v2[:DH], g2[:E])
    y_b = _one_head(q2[DH:], k2[DH:], v2[DH:], g2[E:])
    y2 = jnp.concatenate([y_a, y_b], axis=0)
    wo_j = wo_ref[pl.ds(j * (2 * E * DH), 2 * E * DH), :]
    partial = jax.lax.dot_general(y2.astype(jnp.bfloat16), wo_j,
                                  (((0,), (0,)), ((), ())),
                                  preferred_element_type=jnp.float32)

    @pl.when(j == 0)
    def _():
        o_ref[...] = partial

    @pl.when(j > 0)
    def _():
        o_ref[...] += partial


def kernel(q_src, k_src, v_src, Wq, Wk, Wv, Wo, sel_v, sel_o):
    xq = q_src.reshape(S, D)
    xk = k_src.reshape(S, D)
    xv = v_src.reshape(S, D)
    bf = jnp.bfloat16
    wq = (Wq * SCALE).astype(bf)
    wk = (Wk * SCALE).astype(bf)
    wv_flat = Wv.astype(bf).transpose(1, 0, 2).reshape(D, HE * DH)
    wo_flat = Wo.astype(bf).reshape(HE * DH, D)

    f32 = jnp.float32
    q_t, k_t, v_t, go_t = pl.pallas_call(
        _proj_kernel,
        grid=(S // TS,),
        in_specs=[
            pl.BlockSpec((TS, D), lambda i: (i, 0)),
            pl.BlockSpec((TS, D), lambda i: (i, 0)),
            pl.BlockSpec((TS, D), lambda i: (i, 0)),
            pl.BlockSpec((D, D), lambda i: (0, 0)),
            pl.BlockSpec((D, D), lambda i: (0, 0)),
            pl.BlockSpec((HE, D), lambda i: (0, 0)),
            pl.BlockSpec((HE, D), lambda i: (0, 0)),
            pl.BlockSpec((D, HE * DH), lambda i: (0, 0)),
        ],
        out_specs=[
            pl.BlockSpec((D, TS), lambda i: (0, i)),
            pl.BlockSpec((D, TS), lambda i: (0, i)),
            pl.BlockSpec((D, TS), lambda i: (0, i)),
            pl.BlockSpec((HE, TS), lambda i: (0, i)),
        ],
        out_shape=[
            jax.ShapeDtypeStruct((D, S), bf),
            jax.ShapeDtypeStruct((D, S), bf),
            jax.ShapeDtypeStruct((D, S), bf),
            jax.ShapeDtypeStruct((HE, S), f32),
        ],
        compiler_params=pltpu.CompilerParams(
            dimension_semantics=("parallel",)),
    )(xq, xk, xv, wq, wk, sel_v, sel_o, wv_flat)

    out = pl.pallas_call(
        _attn_out_kernel,
        grid=(S // TQ, H // 2),
        in_specs=[
            pl.BlockSpec((2 * DH, TQ), lambda i, h: (h, i)),
            pl.BlockSpec((2 * DH, S), lambda i, h: (h, 0)),
            pl.BlockSpec((2 * DH, S), lambda i, h: (h, 0)),
            pl.BlockSpec((2 * E, TQ), lambda i, h: (h, i)),
            pl.BlockSpec((HE * DH, D), lambda i, h: (0, 0)),
        ],
        out_specs=pl.BlockSpec((TQ, D), lambda i, h: (i, 0)),
        out_shape=jax.ShapeDtypeStruct((S, D), f32),
        compiler_params=pltpu.CompilerParams(
            dimension_semantics=("parallel", "arbitrary")),
    )(q_t, k_t, v_t, go_t, wo_flat)

    return out.reshape(B, S, D)


# speedup vs baseline: 2.0866x; 1.1541x over previous
import math

import jax
import jax.numpy as jnp
from jax.experimental import pallas as pl
from jax.experimental.pallas import tpu as pltpu

B, S, D = 1, 2048, 1024
H, E, K = 16, 8, 2
DH = D // H
HE = H * E
SCALE = (1.0 / math.sqrt(DH)) ** 0.5

TS = 256
TQ = 256
HB = 4


def _top2_gate_t(logits_t):
    ts = logits_t.shape[-1]
    s = jax.nn.sigmoid(logits_t).reshape(H, E, ts)
    m1 = jnp.max(s, axis=1, keepdims=True)
    eq = s >= m1
    idx = jax.lax.broadcasted_iota(jnp.int32, s.shape, 1)
    first_idx = jnp.min(jnp.where(eq, idx, E), axis=1, keepdims=True)
    s_masked = jnp.where(idx == first_idx, -1.0, s)
    m2 = jnp.max(s_masked, axis=1, keepdims=True)
    return jnp.where(s >= m2, s, 0.0)


def _proj_kernel(xq_ref, xk_ref, xv_ref, wq_ref, wk_ref, sv_ref, so_ref,
                 wv_ref, q_ref, k_ref, v_ref, go_ref):
    xq = xq_ref[...]
    xk = xk_ref[...]
    bf = jnp.bfloat16
    q_ref[...] = jax.lax.dot_general(
        wq_ref[...], xq.astype(bf), (((1,), (1,)), ((), ())),
        preferred_element_type=jnp.float32).astype(bf)
    k_ref[...] = jax.lax.dot_general(
        wk_ref[...], xk.astype(bf), (((1,), (1,)), ((), ())),
        preferred_element_type=jnp.float32).astype(bf)

    lv = jax.lax.dot_general(sv_ref[...], xk, (((1,), (1,)), ((), ())),
                             preferred_element_type=jnp.float32)
    lo = jax.lax.dot_general(so_ref[...], xq, (((1,), (1,)), ((), ())),
                             preferred_element_type=jnp.float32)
    gv = _top2_gate_t(lv)
    go_ref[...] = _top2_gate_t(lo).reshape(HE, TS)

    vall = jax.lax.dot_general(
        wv_ref[...], xv_ref[...].astype(bf), (((0,), (1,)), ((), ())),
        preferred_element_type=jnp.float32)
    v4 = vall.reshape(H, E, DH, TS)
    v_ref[...] = jnp.sum(v4 * gv.reshape(H, E, 1, TS), axis=1
                         ).reshape(H * DH, TS).astype(bf)


def _one_head(q, k, v, g_raw):
    s = jax.lax.dot_general(q, k, (((0,), (0,)), ((), ())),
                            preferred_element_type=jnp.float32)
    m = jnp.max(s, axis=-1, keepdims=True)
    p = jnp.exp(s - m)
    denom = jnp.sum(p, axis=-1, keepdims=True)
    res_t = jax.lax.dot_general(v, p.astype(jnp.bfloat16),
                                (((1,), (1,)), ((), ())),
                                preferred_element_type=jnp.float32)
    g = g_raw * (1.0 / denom).T
    return (g.reshape(E, 1, TQ) * res_t.reshape(1, DH, TQ)).reshape(E * DH, TQ)


def _attn_out_kernel(q_ref, k_ref, v_ref, go_ref, wo_ref, o_ref):
    j = pl.program_id(1)
    q2 = q_ref[...]
    k2 = k_ref[...]
    v2 = v_ref[...]
    g2 = go_ref[...]
    ys = [_one_head(q2[a * DH:(a + 1) * DH], k2[a * DH:(a + 1) * DH],
                    v2[a * DH:(a + 1) * DH], g2[a * E:(a + 1) * E])
          for a in range(HB)]
    y2 = jnp.concatenate(ys, axis=0)
    wo_j = wo_ref[pl.ds(j * (HB * E * DH), HB * E * DH), :]
    partial = jax.lax.dot_general(y2.astype(jnp.bfloat16), wo_j,
                                  (((0,), (0,)), ((), ())),
                                  preferred_element_type=jnp.float32)

    @pl.when(j == 0)
    def _():
        o_ref[...] = partial

    @pl.when(j > 0)
    def _():
        o_ref[...] += partial


def kernel(q_src, k_src, v_src, Wq, Wk, Wv, Wo, sel_v, sel_o):
    xq = q_src.reshape(S, D)
    xk = k_src.reshape(S, D)
    xv = v_src.reshape(S, D)
    bf = jnp.bfloat16
    wq = (Wq * SCALE).astype(bf)
    wk = (Wk * SCALE).astype(bf)
    wv_flat = Wv.astype(bf).transpose(1, 0, 2).reshape(D, HE * DH)
    wo_flat = Wo.astype(bf).reshape(HE * DH, D)

    f32 = jnp.float32
    q_t, k_t, v_t, go_t = pl.pallas_call(
        _proj_kernel,
        grid=(S // TS,),
        in_specs=[
            pl.BlockSpec((TS, D), lambda i: (i, 0)),
            pl.BlockSpec((TS, D), lambda i: (i, 0)),
            pl.BlockSpec((TS, D), lambda i: (i, 0)),
            pl.BlockSpec((D, D), lambda i: (0, 0)),
            pl.BlockSpec((D, D), lambda i: (0, 0)),
            pl.BlockSpec((HE, D), lambda i: (0, 0)),
            pl.BlockSpec((HE, D), lambda i: (0, 0)),
            pl.BlockSpec((D, HE * DH), lambda i: (0, 0)),
        ],
        out_specs=[
            pl.BlockSpec((D, TS), lambda i: (0, i)),
            pl.BlockSpec((D, TS), lambda i: (0, i)),
            pl.BlockSpec((D, TS), lambda i: (0, i)),
            pl.BlockSpec((HE, TS), lambda i: (0, i)),
        ],
        out_shape=[
            jax.ShapeDtypeStruct((D, S), bf),
            jax.ShapeDtypeStruct((D, S), bf),
            jax.ShapeDtypeStruct((D, S), bf),
            jax.ShapeDtypeStruct((HE, S), f32),
        ],
        compiler_params=pltpu.CompilerParams(
            dimension_semantics=("parallel",)),
    )(xq, xk, xv, wq, wk, sel_v, sel_o, wv_flat)

    out = pl.pallas_call(
        _attn_out_kernel,
        grid=(S // TQ, H // HB),
        in_specs=[
            pl.BlockSpec((HB * DH, TQ), lambda i, h: (h, i)),
            pl.BlockSpec((HB * DH, S), lambda i, h: (h, 0)),
            pl.BlockSpec((HB * DH, S), lambda i, h: (h, 0)),
            pl.BlockSpec((HB * E, TQ), lambda i, h: (h, i)),
            pl.BlockSpec((HE * DH, D), lambda i, h: (0, 0)),
        ],
        out_specs=pl.BlockSpec((TQ, D), lambda i, h: (i, 0)),
        out_shape=jax.ShapeDtypeStruct((S, D), f32),
        compiler_params=pltpu.CompilerParams(
            dimension_semantics=("parallel", "arbitrary")),
    )(q_t, k_t, v_t, go_t, wo_flat)

    return out.reshape(B, S, D)


# speedup vs baseline: 2.2411x; 1.0741x over previous
import math

import jax
import jax.numpy as jnp
from jax.experimental import pallas as pl
from jax.experimental.pallas import tpu as pltpu

B, S, D = 1, 2048, 1024
H, E, K = 16, 8, 2
DH = D // H
HE = H * E
SCALE = (1.0 / math.sqrt(DH)) ** 0.5

TS = 256
TQ = 256
HB = 8


def _top2_gate_t(logits_t):
    ts = logits_t.shape[-1]
    s = jax.nn.sigmoid(logits_t).reshape(H, E, ts)
    m1 = jnp.max(s, axis=1, keepdims=True)
    eq = s >= m1
    idx = jax.lax.broadcasted_iota(jnp.int32, s.shape, 1)
    first_idx = jnp.min(jnp.where(eq, idx, E), axis=1, keepdims=True)
    s_masked = jnp.where(idx == first_idx, -1.0, s)
    m2 = jnp.max(s_masked, axis=1, keepdims=True)
    return jnp.where(s >= m2, s, 0.0)


def _proj_kernel(xq_ref, xk_ref, xv_ref, wq_ref, wk_ref, sv_ref, so_ref,
                 wv_ref, q_ref, k_ref, v_ref, go_ref):
    xq = xq_ref[...]
    xk = xk_ref[...]
    bf = jnp.bfloat16
    q_ref[...] = jax.lax.dot_general(
        wq_ref[...], xq.astype(bf), (((1,), (1,)), ((), ())),
        preferred_element_type=jnp.float32).astype(bf)
    k_ref[...] = jax.lax.dot_general(
        wk_ref[...], xk.astype(bf), (((1,), (1,)), ((), ())),
        preferred_element_type=jnp.float32).astype(bf)

    lv = jax.lax.dot_general(sv_ref[...], xk, (((1,), (1,)), ((), ())),
                             preferred_element_type=jnp.float32)
    lo = jax.lax.dot_general(so_ref[...], xq, (((1,), (1,)), ((), ())),
                             preferred_element_type=jnp.float32)
    gv = _top2_gate_t(lv)
    go_ref[...] = _top2_gate_t(lo).reshape(HE, TS)

    vall = jax.lax.dot_general(
        wv_ref[...], xv_ref[...].astype(bf), (((0,), (1,)), ((), ())),
        preferred_element_type=jnp.float32)
    v4 = vall.reshape(H, E, DH, TS)
    v_ref[...] = jnp.sum(v4 * gv.reshape(H, E, 1, TS), axis=1
                         ).reshape(H * DH, TS).astype(bf)


def _one_head(q, k, v, g_raw):
    s = jax.lax.dot_general(q, k, (((0,), (0,)), ((), ())),
                            preferred_element_type=jnp.float32)
    m = jnp.max(s, axis=-1, keepdims=True)
    p = jnp.exp(s - m)
    denom = jnp.sum(p, axis=-1, keepdims=True)
    res_t = jax.lax.dot_general(v, p.astype(jnp.bfloat16),
                                (((1,), (1,)), ((), ())),
                                preferred_element_type=jnp.float32)
    g = g_raw * (1.0 / denom).T
    return (g.reshape(E, 1, TQ) * res_t.reshape(1, DH, TQ)).reshape(E * DH, TQ)


def _attn_out_kernel(q_ref, k_ref, v_ref, go_ref, wo_ref, o_ref):
    j = pl.program_id(1)
    q2 = q_ref[...]
    k2 = k_ref[...]
    v2 = v_ref[...]
    g2 = go_ref[...]
    ys = [_one_head(q2[a * DH:(a + 1) * DH], k2[a * DH:(a + 1) * DH],
                    v2[a * DH:(a + 1) * DH], g2[a * E:(a + 1) * E])
          for a in range(HB)]
    y2 = jnp.concatenate(ys, axis=0)
    wo_j = wo_ref[pl.ds(j * (HB * E * DH), HB * E * DH), :]
    partial = jax.lax.dot_general(y2.astype(jnp.bfloat16), wo_j,
                                  (((0,), (0,)), ((), ())),
                                  preferred_element_type=jnp.float32)

    @pl.when(j == 0)
    def _():
        o_ref[...] = partial

    @pl.when(j > 0)
    def _():
        o_ref[...] += partial


def kernel(q_src, k_src, v_src, Wq, Wk, Wv, Wo, sel_v, sel_o):
    xq = q_src.reshape(S, D)
    xk = k_src.reshape(S, D)
    xv = v_src.reshape(S, D)
    bf = jnp.bfloat16
    wq = (Wq * SCALE).astype(bf)
    wk = (Wk * SCALE).astype(bf)
    wv_flat = Wv.astype(bf).transpose(1, 0, 2).reshape(D, HE * DH)
    wo_flat = Wo.astype(bf).reshape(HE * DH, D)

    f32 = jnp.float32
    q_t, k_t, v_t, go_t = pl.pallas_call(
        _proj_kernel,
        grid=(S // TS,),
        in_specs=[
            pl.BlockSpec((TS, D), lambda i: (i, 0)),
            pl.BlockSpec((TS, D), lambda i: (i, 0)),
            pl.BlockSpec((TS, D), lambda i: (i, 0)),
            pl.BlockSpec((D, D), lambda i: (0, 0)),
            pl.BlockSpec((D, D), lambda i: (0, 0)),
            pl.BlockSpec((HE, D), lambda i: (0, 0)),
            pl.BlockSpec((HE, D), lambda i: (0, 0)),
            pl.BlockSpec((D, HE * DH), lambda i: (0, 0)),
        ],
        out_specs=[
            pl.BlockSpec((D, TS), lambda i: (0, i)),
            pl.BlockSpec((D, TS), lambda i: (0, i)),
            pl.BlockSpec((D, TS), lambda i: (0, i)),
            pl.BlockSpec((HE, TS), lambda i: (0, i)),
        ],
        out_shape=[
            jax.ShapeDtypeStruct((D, S), bf),
            jax.ShapeDtypeStruct((D, S), bf),
            jax.ShapeDtypeStruct((D, S), bf),
            jax.ShapeDtypeStruct((HE, S), f32),
        ],
        compiler_params=pltpu.CompilerParams(
            dimension_semantics=("parallel",)),
    )(xq, xk, xv, wq, wk, sel_v, sel_o, wv_flat)

    out = pl.pallas_call(
        _attn_out_kernel,
        grid=(S // TQ, H // HB),
        in_specs=[
            pl.BlockSpec((HB * DH, TQ), lambda i, h: (h, i)),
            pl.BlockSpec((HB * DH, S), lambda i, h: (h, 0)),
            pl.BlockSpec((HB * DH, S), lambda i, h: (h, 0)),
            pl.BlockSpec((HB * E, TQ), lambda i, h: (h, i)),
            pl.BlockSpec((HE * DH, D), lambda i, h: (0, 0)),
        ],
        out_specs=pl.BlockSpec((TQ, D), lambda i, h: (i, 0)),
        out_shape=jax.ShapeDtypeStruct((S, D), f32),
        compiler_params=pltpu.CompilerParams(
            dimension_semantics=("parallel", "arbitrary")),
    )(q_t, k_t, v_t, go_t, wo_flat)

    return out.reshape(B, S, D)


# speedup vs baseline: 2.2932x; 1.0233x over previous
import math

import jax
import jax.numpy as jnp
from jax.experimental import pallas as pl
from jax.experimental.pallas import tpu as pltpu

B, S, D = 1, 2048, 1024
H, E, K = 16, 8, 2
DH = D // H
HE = H * E
SCALE = (1.0 / math.sqrt(DH)) ** 0.5

TS = 512
TQ = 256
HB = 8


def _top2_gate_t(logits_t):
    ts = logits_t.shape[-1]
    s = jax.nn.sigmoid(logits_t).reshape(H, E, ts)
    m1 = jnp.max(s, axis=1, keepdims=True)
    eq = s >= m1
    idx = jax.lax.broadcasted_iota(jnp.int32, s.shape, 1)
    first_idx = jnp.min(jnp.where(eq, idx, E), axis=1, keepdims=True)
    s_masked = jnp.where(idx == first_idx, -1.0, s)
    m2 = jnp.max(s_masked, axis=1, keepdims=True)
    return jnp.where(s >= m2, s, 0.0)


def _proj_kernel(xq_ref, xk_ref, xv_ref, wq_ref, wk_ref, sv_ref, so_ref,
                 wv_ref, q_ref, k_ref, v_ref, go_ref):
    xq = xq_ref[...]
    xk = xk_ref[...]
    bf = jnp.bfloat16
    q_ref[...] = jax.lax.dot_general(
        wq_ref[...], xq.astype(bf), (((1,), (1,)), ((), ())),
        preferred_element_type=jnp.float32).astype(bf)
    k_ref[...] = jax.lax.dot_general(
        wk_ref[...], xk.astype(bf), (((1,), (1,)), ((), ())),
        preferred_element_type=jnp.float32).astype(bf)

    lv = jax.lax.dot_general(sv_ref[...], xk, (((1,), (1,)), ((), ())),
                             preferred_element_type=jnp.float32)
    lo = jax.lax.dot_general(so_ref[...], xq, (((1,), (1,)), ((), ())),
                             preferred_element_type=jnp.float32)
    gv = _top2_gate_t(lv)
    go_ref[...] = _top2_gate_t(lo).reshape(HE, TS)

    vall = jax.lax.dot_general(
        wv_ref[...], xv_ref[...].astype(bf), (((0,), (1,)), ((), ())),
        preferred_element_type=jnp.float32)
    v4 = vall.reshape(H, E, DH, TS)
    v_ref[...] = jnp.sum(v4 * gv.reshape(H, E, 1, TS), axis=1
                         ).reshape(H * DH, TS).astype(bf)


def _one_head(q, k, v, g_raw):
    s = jax.lax.dot_general(q, k, (((0,), (0,)), ((), ())),
                            preferred_element_type=jnp.float32)
    m = jnp.max(s, axis=-1, keepdims=True)
    p = jnp.exp(s - m)
    denom = jnp.sum(p, axis=-1, keepdims=True)
    res_t = jax.lax.dot_general(v, p.astype(jnp.bfloat16),
                                (((1,), (1,)), ((), ())),
                                preferred_element_type=jnp.float32)
    g = g_raw * (1.0 / denom).T
    return (g.reshape(E, 1, TQ) * res_t.reshape(1, DH, TQ)).reshape(E * DH, TQ)


def _attn_out_kernel(q_ref, k_ref, v_ref, go_ref, wo_ref, o_ref):
    j = pl.program_id(1)
    q2 = q_ref[...]
    k2 = k_ref[...]
    v2 = v_ref[...]
    g2 = go_ref[...]
    ys = [_one_head(q2[a * DH:(a + 1) * DH], k2[a * DH:(a + 1) * DH],
                    v2[a * DH:(a + 1) * DH], g2[a * E:(a + 1) * E])
          for a in range(HB)]
    y2 = jnp.concatenate(ys, axis=0)
    wo_j = wo_ref[pl.ds(j * (HB * E * DH), HB * E * DH), :]
    partial = jax.lax.dot_general(y2.astype(jnp.bfloat16), wo_j,
                                  (((0,), (0,)), ((), ())),
                                  preferred_element_type=jnp.float32)

    @pl.when(j == 0)
    def _():
        o_ref[...] = partial

    @pl.when(j > 0)
    def _():
        o_ref[...] += partial


def kernel(q_src, k_src, v_src, Wq, Wk, Wv, Wo, sel_v, sel_o):
    xq = q_src.reshape(S, D)
    xk = k_src.reshape(S, D)
    xv = v_src.reshape(S, D)
    bf = jnp.bfloat16
    wq = (Wq * SCALE).astype(bf)
    wk = (Wk * SCALE).astype(bf)
    wv_flat = Wv.transpose(1, 0, 2).astype(bf).reshape(D, HE * DH)
    wo_flat = Wo.astype(bf).reshape(HE * DH, D)

    f32 = jnp.float32
    q_t, k_t, v_t, go_t = pl.pallas_call(
        _proj_kernel,
        grid=(S // TS,),
        in_specs=[
            pl.BlockSpec((TS, D), lambda i: (i, 0)),
            pl.BlockSpec((TS, D), lambda i: (i, 0)),
            pl.BlockSpec((TS, D), lambda i: (i, 0)),
            pl.BlockSpec((D, D), lambda i: (0, 0)),
            pl.BlockSpec((D, D), lambda i: (0, 0)),
            pl.BlockSpec((HE, D), lambda i: (0, 0)),
            pl.BlockSpec((HE, D), lambda i: (0, 0)),
            pl.BlockSpec((D, HE * DH), lambda i: (0, 0)),
        ],
        out_specs=[
            pl.BlockSpec((D, TS), lambda i: (0, i)),
            pl.BlockSpec((D, TS), lambda i: (0, i)),
            pl.BlockSpec((D, TS), lambda i: (0, i)),
            pl.BlockSpec((HE, TS), lambda i: (0, i)),
        ],
        out_shape=[
            jax.ShapeDtypeStruct((D, S), bf),
            jax.ShapeDtypeStruct((D, S), bf),
            jax.ShapeDtypeStruct((D, S), bf),
            jax.ShapeDtypeStruct((HE, S), f32),
        ],
        compiler_params=pltpu.CompilerParams(
            dimension_semantics=("parallel",)),
    )(xq, xk, xv, wq, wk, sel_v, sel_o, wv_flat)

    out = pl.pallas_call(
        _attn_out_kernel,
        grid=(S // TQ, H // HB),
        in_specs=[
            pl.BlockSpec((HB * DH, TQ), lambda i, h: (h, i)),
            pl.BlockSpec((HB * DH, S), lambda i, h: (h, 0)),
            pl.BlockSpec((HB * DH, S), lambda i, h: (h, 0)),
            pl.BlockSpec((HB * E, TQ), lambda i, h: (h, i)),
            pl.BlockSpec((HE * DH, D), lambda i, h: (0, 0)),
        ],
        out_specs=pl.BlockSpec((TQ, D), lambda i, h: (i, 0)),
        out_shape=jax.ShapeDtypeStruct((S, D), f32),
        compiler_params=pltpu.CompilerParams(
            dimension_semantics=("parallel", "arbitrary")),
    )(q_t, k_t, v_t, go_t, wo_flat)

    return out.reshape(B, S, D)
